# baseline (speedup 1.0000x reference)
import functools

import jax
import jax.numpy as jnp
from jax import lax
from jax.experimental import pallas as pl
from jax.experimental.pallas import tpu as pltpu
from jax.experimental.pallas import tpu_sc as plsc

TEMP_INV = 10.0

B = 16384
C0 = 64
C1 = 256
D = 32

NC, NS = 2, 16
NW = NC * NS
BPW = B // NW

A0_CHUNK = 128
A1_CHUNK = 64
IT_CHUNK = 128
A0_NCHUNK = BPW // A0_CHUNK
A1_NCHUNK = BPW // A1_CHUNK
IT_NCHUNK = BPW // IT_CHUNK


def _sc_gather(uid2, uid, iid4, a0r, a1, itr):
    mesh = plsc.VectorSubcoreMesh(core_axis_name="c", subcore_axis_name="s")

    @functools.partial(
        pl.kernel,
        mesh=mesh,
        out_type=(
            jax.ShapeDtypeStruct((B, 128), jnp.float32),
            jax.ShapeDtypeStruct((B, C1), jnp.float32),
            jax.ShapeDtypeStruct((B, 128), jnp.float32),
        ),
        scratch_types=[
            pltpu.VMEM((A0_NCHUNK, A0_CHUNK), jnp.int32),
            pltpu.VMEM((A1_NCHUNK, A1_CHUNK), jnp.int32),
            pltpu.VMEM((IT_NCHUNK, IT_CHUNK), jnp.int32),
            pltpu.VMEM((A0_CHUNK, 128), jnp.float32),
            pltpu.VMEM((A0_CHUNK, 128), jnp.float32),
            pltpu.VMEM((A1_CHUNK, C1), jnp.float32),
            pltpu.VMEM((A1_CHUNK, C1), jnp.float32),
            pltpu.VMEM((IT_CHUNK, 128), jnp.float32),
            pltpu.VMEM((IT_CHUNK, 128), jnp.float32),
            pltpu.SemaphoreType.DMA,
            pltpu.SemaphoreType.DMA,
            pltpu.SemaphoreType.DMA,
            pltpu.SemaphoreType.DMA,
            pltpu.SemaphoreType.DMA,
            pltpu.SemaphoreType.DMA,
        ],
    )
    def k(uid2_hbm, uid_hbm, iid4_hbm, a0_hbm, a1_hbm, it_hbm,
          g0_hbm, g1_hbm, v_hbm,
          uidx0_v, uidx1_v, iidx_v, a0_p, a0_q, a1_p, a1_q, it_p, it_q,
          s0p, s0q, s1p, s1q, sip, siq):
        wid = lax.axis_index("s") * NC + lax.axis_index("c")
        base = wid * BPW
        for j in range(A0_NCHUNK):
            pltpu.sync_copy(
                uid2_hbm.at[pl.ds(base + j * A0_CHUNK, A0_CHUNK)],
                uidx0_v.at[j])
        for j in range(A1_NCHUNK):
            pltpu.sync_copy(
                uid_hbm.at[pl.ds(base + j * A1_CHUNK, A1_CHUNK)],
                uidx1_v.at[j])
        for j in range(IT_NCHUNK):
            pltpu.sync_copy(
                iid4_hbm.at[pl.ds(base + j * IT_CHUNK, IT_CHUNK)],
                iidx_v.at[j])

        jobs = (
            (a0_hbm, uidx0_v, A0_CHUNK, A0_NCHUNK, (a0_p, a0_q), (s0p, s0q),
             g0_hbm),
            (a1_hbm, uidx1_v, A1_CHUNK, A1_NCHUNK, (a1_p, a1_q), (s1p, s1q),
             g1_hbm),
            (it_hbm, iidx_v, IT_CHUNK, IT_NCHUNK, (it_p, it_q), (sip, siq),
             v_hbm),
        )

        def fire(job, kk):
            src, idx, chunk, _, bufs, sems, _ = job
            return pltpu.async_copy(
                src.at[idx.at[kk]],
                bufs[kk % 2], sems[kk % 2])

        cps = [[None] * j[3] for j in jobs]
        for ji, job in enumerate(jobs):
            for kk in range(min(2, job[3])):
                cps[ji][kk] = fire(job, kk)

        for rnd in range(max(j[3] for j in jobs)):
            for ji, job in enumerate(jobs):
                src, idx, chunk, nchunk, bufs, sems, out = job
                if rnd >= nchunk:
                    continue
                cps[ji][rnd].wait()
                pltpu.sync_copy(
                    bufs[rnd % 2],
                    out.at[pl.ds(base + rnd * chunk, chunk)])
                nxt = rnd + 2
                if nxt < nchunk:
                    cps[ji][nxt] = fire(job, nxt)

    return k(uid2, uid, iid4, a0r, a1, itr)


def _tc_body(par_ref, sel_ref, g0_ref, g1_ref, v_ref, c0_ref, c1_ref, o_ref):
    par = par_ref[...] == 0
    l0 = jnp.where(par, g0_ref[:, :C0], g0_ref[:, C0:]) * TEMP_INV
    l0 = l0 - jnp.max(l0, axis=1, keepdims=True)
    e0 = jnp.exp(l0)
    w0 = e0 / jnp.sum(e0, axis=1, keepdims=True)

    l1 = g1_ref[...] * TEMP_INV
    l1 = l1 - jnp.max(l1, axis=1, keepdims=True)
    e1 = jnp.exp(l1)
    w1 = e1 / jnp.sum(e1, axis=1, keepdims=True)

    u = (jnp.dot(w0, c0_ref[...], preferred_element_type=jnp.float32)
         + jnp.dot(w1, c1_ref[...], preferred_element_type=jnp.float32))

    sel = sel_ref[...]
    v = jnp.where(
        sel < 2,
        jnp.where(sel == 0, v_ref[:, 0:D], v_ref[:, D:2 * D]),
        jnp.where(sel == 2, v_ref[:, 2 * D:3 * D], v_ref[:, 3 * D:]))
    o_ref[...] = jnp.sum(u * v, axis=1, keepdims=True)


def _tc_compute(par, sel, g0, g1, v, codebook0, codebook1):
    TB = 2048
    grid = (B // TB,)
    return pl.pallas_call(
        _tc_body,
        grid=grid,
        in_specs=[
            pl.BlockSpec((TB, 1), lambda i: (i, 0)),
            pl.BlockSpec((TB, 1), lambda i: (i, 0)),
            pl.BlockSpec((TB, 128), lambda i: (i, 0)),
            pl.BlockSpec((TB, C1), lambda i: (i, 0)),
            pl.BlockSpec((TB, 128), lambda i: (i, 0)),
            pl.BlockSpec((C0, D), lambda i: (0, 0)),
            pl.BlockSpec((C1, D), lambda i: (0, 0)),
        ],
        out_specs=pl.BlockSpec((TB, 1), lambda i: (i, 0)),
        out_shape=jax.ShapeDtypeStruct((B, 1), jnp.float32),
    )(par, sel, g0, g1, v, codebook0, codebook1)


def kernel(X, assign0, codebook0, assign1, codebook1, item_table):
    uid = X[:, 0]
    iid = X[:, 1]
    uid2 = lax.shift_right_logical(uid, 1)
    iid4 = lax.shift_right_logical(iid, 2)
    par = (uid & 1).reshape(B, 1)
    sel = (iid & 3).reshape(B, 1)
    a0r = assign0.reshape(assign0.shape[0] // 2, 128)
    itr = item_table.reshape(item_table.shape[0] // 4, 128)
    g0, g1, v = _sc_gather(uid2, uid, iid4, a0r, assign1, itr)
    return _tc_compute(par, sel, g0, g1, v, codebook0, codebook1)
